# baseline (speedup 1.0000x reference)
import functools

import jax
import jax.numpy as jnp
from jax.experimental import pallas as pl
from jax.experimental.pallas import tpu as pltpu


def _decode_block(params_ref, x_ref, o_ref, y_ref, sems, *, nG, nCH, nA, nB):
    b = pl.program_id(0)
    par = jax.lax.rem(b, 2)
    n = nG * nG
    stride = params_ref[2 * nA]
    clipmax = params_ref[2 * nA + 1]

    def _dma(src_b, buf):
        return pltpu.make_async_copy(
            y_ref.at[buf], o_ref.at[src_b], sems.at[buf]
        )

    @pl.when(b >= 2)
    def _():
        _dma(b - 2, par).wait()

    x = x_ref[0].reshape(n, nA * nCH)
    xt = x.T

    g = jax.lax.broadcasted_iota(jnp.int32, (1, n), 1)
    gx = (g % nG).astype(jnp.float32)
    gy = (g // nG).astype(jnp.float32)
    cx = (gx + 0.5) * stride
    cy = (gy + 0.5) * stride

    for a in range(nA):
        base = a * nCH
        aw = params_ref[2 * a]
        ah = params_ref[2 * a + 1]
        e = jnp.exp(xt[base : base + 4, :])
        l = jnp.clip(e[0:1] * aw, 0.0, clipmax)
        t = jnp.clip(e[1:2] * ah, 0.0, clipmax)
        r = jnp.clip(e[2:3] * aw, 0.0, clipmax)
        bb = jnp.clip(e[3:4] * ah, 0.0, clipmax)
        xc = cx + (r - l) * 0.5
        yc = cy + (bb - t) * 0.5
        w = l + r
        h = t + bb
        sig = jax.nn.sigmoid(xt[base + 4 : base + nCH, :])
        y_ref[par, :, a * n : (a + 1) * n] = jnp.concatenate(
            [xc, yc, w, h, sig], axis=0
        )

    _dma(b, par).start()

    @pl.when(b == nB - 1)
    def _():
        if nB >= 2:
            _dma(b - 1, 1 - par).wait()
        _dma(b, par).wait()


def kernel(raw, anchors, img_size):
    nB, C, nG, _ = raw.shape
    nA = anchors.shape[0]
    nCH = C // nA
    img = jnp.asarray(img_size)
    stride = (img // nG).astype(jnp.float32)
    clipmax = img.astype(jnp.float32)
    params = jnp.concatenate(
        [anchors.reshape(-1).astype(jnp.float32), jnp.stack([stride, clipmax])]
    )
    x = jnp.transpose(raw, (0, 2, 3, 1))
    out = pl.pallas_call(
        functools.partial(_decode_block, nG=nG, nCH=nCH, nA=nA, nB=nB),
        grid=(nB,),
        in_specs=[
            pl.BlockSpec(memory_space=pltpu.SMEM),
            pl.BlockSpec((1, nG, nG, C), lambda b: (b, 0, 0, 0)),
        ],
        out_specs=pl.BlockSpec(memory_space=pl.ANY),
        out_shape=jax.ShapeDtypeStruct((nB, nCH, nA * nG * nG), jnp.float32),
        scratch_shapes=[
            pltpu.VMEM((2, nCH, nA * nG * nG), jnp.float32),
            pltpu.SemaphoreType.DMA((2,)),
        ],
    )(params, x)
    return jnp.transpose(out, (0, 2, 1))


# speedup vs baseline: 2.3039x; 2.3039x over previous
import functools

import jax
import jax.numpy as jnp
from jax.experimental import pallas as pl
from jax.experimental.pallas import tpu as pltpu


def _decode_block(params_ref, x_ref, o_ref, y_ref, sems, *, nG, nCH, nA, nB):
    b = pl.program_id(0)
    par = jax.lax.rem(b, 3)
    n = nG * nG
    stride = params_ref[2 * nA]
    clipmax = params_ref[2 * nA + 1]

    def _dma(src_b, buf):
        return pltpu.make_async_copy(
            y_ref.at[buf], o_ref.at[:, src_b, :], sems.at[buf]
        )

    @pl.when(b >= 3)
    def _():
        _dma(b - 3, par).wait()

    x = x_ref[0].reshape(n, nA * nCH)
    xt = x.T

    g = jax.lax.broadcasted_iota(jnp.int32, (1, n), 1)
    gx = (g % nG).astype(jnp.float32)
    gy = (g // nG).astype(jnp.float32)
    cx = (gx + 0.5) * stride
    cy = (gy + 0.5) * stride

    for a in range(nA):
        base = a * nCH
        aw = params_ref[2 * a]
        ah = params_ref[2 * a + 1]
        e = jnp.exp(xt[base : base + 4, :])
        l = jnp.clip(e[0:1] * aw, 0.0, clipmax)
        t = jnp.clip(e[1:2] * ah, 0.0, clipmax)
        r = jnp.clip(e[2:3] * aw, 0.0, clipmax)
        bb = jnp.clip(e[3:4] * ah, 0.0, clipmax)
        xc = cx + (r - l) * 0.5
        yc = cy + (bb - t) * 0.5
        w = l + r
        h = t + bb
        sig = jax.nn.sigmoid(xt[base + 4 : base + nCH, :])
        y_ref[par, :, a * n : (a + 1) * n] = jnp.concatenate(
            [xc, yc, w, h, sig], axis=0
        )

    _dma(b, par).start()

    @pl.when(b == nB - 1)
    def _():
        if nB >= 3:
            _dma(b - 2, jax.lax.rem(b + 1, 3)).wait()
        if nB >= 2:
            _dma(b - 1, jax.lax.rem(b + 2, 3)).wait()
        _dma(b, par).wait()


def kernel(raw, anchors, img_size):
    nB, C, nG, _ = raw.shape
    nA = anchors.shape[0]
    nCH = C // nA
    img = jnp.asarray(img_size)
    stride = (img // nG).astype(jnp.float32)
    clipmax = img.astype(jnp.float32)
    params = jnp.concatenate(
        [anchors.reshape(-1).astype(jnp.float32), jnp.stack([stride, clipmax])]
    )
    x = jnp.transpose(raw, (0, 2, 3, 1))
    out = pl.pallas_call(
        functools.partial(_decode_block, nG=nG, nCH=nCH, nA=nA, nB=nB),
        grid=(nB,),
        in_specs=[
            pl.BlockSpec(memory_space=pltpu.SMEM),
            pl.BlockSpec((1, nG, nG, C), lambda b: (b, 0, 0, 0)),
        ],
        out_specs=pl.BlockSpec(memory_space=pl.ANY),
        out_shape=jax.ShapeDtypeStruct((nCH, nB, nA * nG * nG), jnp.float32),
        scratch_shapes=[
            pltpu.VMEM((3, nCH, nA * nG * nG), jnp.float32),
            pltpu.SemaphoreType.DMA((3,)),
        ],
    )(params, x)
    return jnp.transpose(out, (1, 2, 0))


# speedup vs baseline: 2.3041x; 1.0001x over previous
import functools

import jax
import jax.numpy as jnp
from jax.experimental import pallas as pl
from jax.experimental.pallas import tpu as pltpu


def _decode_block(params_ref, x_ref, o_ref, y_ref, sems, *, nG, nCH, nA, nB):
    b = pl.program_id(0)
    par = jax.lax.rem(b, 2)
    n = nG * nG
    stride = params_ref[2 * nA]
    clipmax = params_ref[2 * nA + 1]

    def _dma(src_b, buf):
        return pltpu.make_async_copy(
            y_ref.at[buf], o_ref.at[:, src_b, :], sems.at[buf]
        )

    @pl.when(b >= 2)
    def _():
        _dma(b - 2, par).wait()

    x = x_ref[0].reshape(n, nA * nCH)
    xt = x.T

    g = jax.lax.broadcasted_iota(jnp.int32, (1, n), 1)
    gx = (g % nG).astype(jnp.float32)
    gy = (g // nG).astype(jnp.float32)
    cx = (gx + 0.5) * stride
    cy = (gy + 0.5) * stride

    for a in range(nA):
        base = a * nCH
        aw = params_ref[2 * a]
        ah = params_ref[2 * a + 1]
        e = jnp.exp(xt[base : base + 4, :])
        l = jnp.clip(e[0:1] * aw, 0.0, clipmax)
        t = jnp.clip(e[1:2] * ah, 0.0, clipmax)
        r = jnp.clip(e[2:3] * aw, 0.0, clipmax)
        bb = jnp.clip(e[3:4] * ah, 0.0, clipmax)
        xc = cx + (r - l) * 0.5
        yc = cy + (bb - t) * 0.5
        w = l + r
        h = t + bb
        sig = jax.nn.sigmoid(xt[base + 4 : base + nCH, :])
        y_ref[par, :, a * n : (a + 1) * n] = jnp.concatenate(
            [xc, yc, w, h, sig], axis=0
        )

    _dma(b, par).start()

    @pl.when(b == nB - 1)
    def _():
        if nB >= 2:
            _dma(b - 1, 1 - par).wait()
        _dma(b, par).wait()


def kernel(raw, anchors, img_size):
    nB, C, nG, _ = raw.shape
    nA = anchors.shape[0]
    nCH = C // nA
    img = jnp.asarray(img_size)
    stride = (img // nG).astype(jnp.float32)
    clipmax = img.astype(jnp.float32)
    params = jnp.concatenate(
        [anchors.reshape(-1).astype(jnp.float32), jnp.stack([stride, clipmax])]
    )
    x = jnp.transpose(raw, (0, 2, 3, 1))
    out = pl.pallas_call(
        functools.partial(_decode_block, nG=nG, nCH=nCH, nA=nA, nB=nB),
        grid=(nB,),
        in_specs=[
            pl.BlockSpec(memory_space=pltpu.SMEM),
            pl.BlockSpec((1, nG, nG, C), lambda b: (b, 0, 0, 0)),
        ],
        out_specs=pl.BlockSpec(memory_space=pl.ANY),
        out_shape=jax.ShapeDtypeStruct((nCH, nB, nA * nG * nG), jnp.float32),
        scratch_shapes=[
            pltpu.VMEM((2, nCH, nA * nG * nG), jnp.float32),
            pltpu.SemaphoreType.DMA((2,)),
        ],
    )(params, x)
    return jnp.transpose(out, (1, 2, 0))
